# baseline (speedup 1.0000x reference)
import functools

import jax
import jax.numpy as jnp
from jax import lax
from jax.experimental import pallas as pl
from jax.experimental.pallas import tpu as pltpu
from jax.experimental.pallas import tpu_sc as plsc

N_NODES = 50000
N_EDGES = 800000
HIDDEN = 64

NC = 2
NS = 16
NW = NC * NS

IROW = 125
NROWS = N_EDGES // IROW
EPW_G = N_EDGES // NW
RPW_G = NROWS // NW
GROWS = 8
GCHUNK = GROWS * IROW
HPAD = 128
EPT_S = N_EDGES // NS
RPT_S = NROWS // NS
SROWS = 2
SCHUNK = SROWS * IROW
SITER = EPT_S // (2 * SCHUNK)
NPT = N_NODES // NS
FH = 32


@functools.lru_cache(maxsize=1)
def _mesh():
    return plsc.VectorSubcoreMesh(core_axis_name="c", subcore_axis_name="s",
                                  num_cores=NC, num_subcores=NS)


def _gather_body(h_hbm, s2_hbm, e2_hbm, hs_hbm, he_hbm, idx_v, rows_v, sem):
    c = lax.axis_index("c")
    s = lax.axis_index("s")
    wid = s * NC + c
    for idx_hbm, out_hbm in ((s2_hbm, hs_hbm), (e2_hbm, he_hbm)):
        def chunk(i, carry, idx_hbm=idx_hbm, out_hbm=out_hbm):
            rbase = wid * RPW_G + i * GROWS
            ebase = wid * EPW_G + i * GCHUNK
            pltpu.sync_copy(idx_hbm.at[pl.ds(rbase, GROWS)], idx_v)
            cps = [
                pltpu.async_copy(h_hbm.at[idx_v.at[j]],
                                 rows_v.at[pl.ds(j * IROW, IROW)], sem)
                for j in range(GROWS)
            ]
            for cp in cps:
                cp.wait()
            pltpu.sync_copy(rows_v, out_hbm.at[pl.ds(ebase, GCHUNK)])
            return carry
        lax.fori_loop(0, EPW_G // GCHUNK, chunk, 0)


def _sc_gather(h, s2, e2):
    return pl.kernel(
        _gather_body,
        out_type=[jax.ShapeDtypeStruct((N_EDGES, HPAD), jnp.float32),
                  jax.ShapeDtypeStruct((N_EDGES, HPAD), jnp.float32)],
        mesh=_mesh(),
        scratch_types=[
            pltpu.VMEM((GROWS, IROW), jnp.int32),
            pltpu.VMEM((GCHUNK, HPAD), jnp.float32),
            pltpu.SemaphoreType.DMA,
        ],
    )(h, s2, e2)


def _scatter_body(e_hbm, s2_hbm, e2_hbm, z_hbm, agg_e_hbm, agg_s_hbm,
                  idx0, idx1, val0, val1, acc, sl0, sl1, ss0, ss1):
    c = lax.axis_index("c")
    s = lax.axis_index("s")
    rb = s * NPT
    slots = ((idx0, val0, sl0, ss0), (idx1, val1, sl1, ss1))

    def do_half(fh):
        for idx_hbm, agg_hbm in ((e2_hbm, agg_e_hbm), (s2_hbm, agg_s_hbm)):
            pltpu.sync_copy(z_hbm.at[pl.ds(rb, NPT)], acc.at[pl.ds(rb, NPT)])
            plsc.subcore_barrier()

            def step(g, carry, idx_hbm=idx_hbm):
                for b, (idx_v, val_v, sl, ss) in enumerate(slots):
                    i = 2 * g + b
                    rbase = s * RPT_S + i * SROWS
                    ebase = s * EPT_S + i * SCHUNK

                    @pl.when(g > 0)
                    def _(idx_v=idx_v, val_v=val_v, ss=ss):
                        for j in range(SROWS):
                            pltpu.make_async_copy(
                                val_v.at[pl.ds(j * IROW, IROW)],
                                acc.at[idx_v.at[j]], ss).wait()

                    pltpu.async_copy(idx_hbm.at[pl.ds(rbase, SROWS)],
                                     idx_v, sl)
                    pltpu.async_copy(
                        e_hbm.at[pl.ds(ebase, SCHUNK), pl.ds(fh, FH)],
                        val_v, sl)
                for b, (idx_v, val_v, sl, ss) in enumerate(slots):
                    pltpu.make_async_copy(
                        idx_hbm.at[pl.ds(s * RPT_S, SROWS)], idx_v, sl).wait()
                    pltpu.make_async_copy(
                        e_hbm.at[pl.ds(s * EPT_S, SCHUNK), pl.ds(fh, FH)],
                        val_v, sl).wait()
                    for j in range(SROWS):
                        pltpu.async_copy(val_v.at[pl.ds(j * IROW, IROW)],
                                         acc.at[idx_v.at[j]], ss, add=True)
                return carry
            lax.fori_loop(0, SITER, step, 0)
            for b, (idx_v, val_v, sl, ss) in enumerate(slots):
                for j in range(SROWS):
                    pltpu.make_async_copy(
                        val_v.at[pl.ds(j * IROW, IROW)],
                        acc.at[idx_v.at[j]], ss).wait()
            plsc.subcore_barrier()
            pltpu.sync_copy(acc.at[pl.ds(rb, NPT)],
                            agg_hbm.at[pl.ds(rb, NPT), pl.ds(fh, FH)])
            plsc.subcore_barrier()

    for cc in range(NC):
        @pl.when(c == cc)
        def _(cc=cc):
            do_half(cc * FH)


def _sc_segment_sums(e, s2, e2, z32):
    return pl.kernel(
        _scatter_body,
        out_type=[jax.ShapeDtypeStruct((N_NODES, HIDDEN), jnp.float32),
                  jax.ShapeDtypeStruct((N_NODES, HIDDEN), jnp.float32)],
        mesh=_mesh(),
        scratch_types=[
            pltpu.VMEM((SROWS, IROW), jnp.int32),
            pltpu.VMEM((SROWS, IROW), jnp.int32),
            pltpu.VMEM((SCHUNK, FH), jnp.float32),
            pltpu.VMEM((SCHUNK, FH), jnp.float32),
            pltpu.VMEM_SHARED((N_NODES, FH), jnp.float32),
            pltpu.SemaphoreType.DMA,
            pltpu.SemaphoreType.DMA,
            pltpu.SemaphoreType.DMA,
            pltpu.SemaphoreType.DMA,
        ],
        compiler_params=pltpu.CompilerParams(use_tc_tiling_on_sc=False),
    )(e, s2, e2, z32)


def _tc_mlp(inputs, layers, use_dims, acts, block_rows, n_rows,
            out_pad=None):
    flat = []
    for lp in layers:
        flat.append(lp[0])
        flat.append(lp[1].reshape(1, -1))
        if len(lp) == 4:
            flat.append(lp[2].reshape(1, -1))
            flat.append(lp[3].reshape(1, -1))
    out_dim = layers[-1][0].shape[1]
    phys_dims = [a.shape[1] for a in inputs]
    n_in = len(inputs)
    out_phys = out_pad if out_pad is not None else out_dim

    def body(*refs):
        irefs = refs[:n_in]
        wrefs = refs[n_in:-1]
        oref = refs[-1]
        wi = 0
        xcur = None
        for li, lp in enumerate(layers):
            w = wrefs[wi][...]
            b = wrefs[wi + 1][...]
            wi += 2
            if li == 0:
                off = 0
                z = None
                for k, ir in enumerate(irefs):
                    xk = ir[...][:, :use_dims[k]]
                    t = jnp.dot(xk.astype(jnp.bfloat16),
                                w[off:off + use_dims[k], :].astype(
                                    jnp.bfloat16),
                                preferred_element_type=jnp.float32)
                    z = t if z is None else z + t
                    off += use_dims[k]
                z = z + b
            else:
                z = jnp.dot(xcur.astype(jnp.bfloat16),
                            w.astype(jnp.bfloat16),
                            preferred_element_type=jnp.float32) + b
            if len(lp) == 4:
                g = wrefs[wi][...]
                bt = wrefs[wi + 1][...]
                wi += 2
                d = z.shape[-1]
                ones_w = jnp.full((d, 1), 1.0 / d, jnp.float32)
                mu = jnp.dot(z, ones_w, preferred_element_type=jnp.float32)
                zc = z - mu
                var = jnp.dot(zc * zc, ones_w,
                              preferred_element_type=jnp.float32)
                z = zc * lax.rsqrt(var + 1e-5) * g + bt
            if acts[li] == 'relu':
                z = jnp.maximum(z, 0.0)
            elif acts[li] == 'tanh':
                z = jnp.tanh(z)
            xcur = z
        if out_phys > out_dim:
            pad = jnp.zeros((xcur.shape[0], out_phys - out_dim), jnp.float32)
            xcur = jnp.concatenate([xcur, pad], axis=-1)
        oref[...] = xcur

    grid = (n_rows // block_rows,)
    in_specs = (
        [pl.BlockSpec((block_rows, d), lambda i: (i, 0)) for d in phys_dims]
        + [pl.BlockSpec(w.shape, lambda i: (0,) * w.ndim) for w in flat]
    )
    return pl.pallas_call(
        body,
        grid=grid,
        in_specs=in_specs,
        out_specs=pl.BlockSpec((block_rows, out_phys), lambda i: (i, 0)),
        out_shape=jax.ShapeDtypeStruct((n_rows, out_phys), jnp.float32),
        compiler_params=pltpu.CompilerParams(
            dimension_semantics=("arbitrary",)),
    )(*inputs, *flat)


B_NODE = 2000
B_EDGE = 8000


def kernel(x, edge_index, node_enc, edge_enc, node_net, edge_net, edge_clf):
    start = edge_index[0]
    end = edge_index[1]
    s2 = start.reshape(NROWS, IROW)
    e2 = end.reshape(NROWS, IROW)
    z32 = jnp.zeros((N_NODES, FH), jnp.float32)

    h = _tc_mlp([x], node_enc, [3], ['relu', 'tanh'], B_NODE, N_NODES,
                out_pad=HPAD)
    hs, he = _sc_gather(h, s2, e2)
    e = _tc_mlp([hs, he], edge_enc, [HIDDEN, HIDDEN], ['relu', 'tanh'],
                B_EDGE, N_EDGES)
    for _ in range(3):
        agg_e, agg_s = _sc_segment_sums(e, s2, e2, z32)
        h = _tc_mlp([h, agg_e, agg_s], node_net, [HIDDEN] * 3,
                    ['relu', 'tanh'], B_NODE, N_NODES, out_pad=HPAD)
        hs, he = _sc_gather(h, s2, e2)
        e = _tc_mlp([hs, he, e], edge_net, [HIDDEN] * 3, ['relu', 'tanh'],
                    B_EDGE, N_EDGES)
    out = _tc_mlp([hs, he, e], edge_clf, [HIDDEN] * 3,
                  ['relu', 'relu', None], B_EDGE, N_EDGES)
    return jnp.squeeze(out, axis=-1)


# speedup vs baseline: 1.0428x; 1.0428x over previous
import functools

import jax
import jax.numpy as jnp
from jax import lax
from jax.experimental import pallas as pl
from jax.experimental.pallas import tpu as pltpu
from jax.experimental.pallas import tpu_sc as plsc

N_NODES = 50000
N_EDGES = 800000
HIDDEN = 64

NC = 2
NS = 16
NW = NC * NS

IROW = 125
NROWS = N_EDGES // IROW
EPW_G = N_EDGES // NW
RPW_G = NROWS // NW
GROWS = 8
GCHUNK = GROWS * IROW
HPAD = 128
EPT_S = N_EDGES // NS
RPT_S = NROWS // NS
SROWS = 2
SCHUNK = SROWS * IROW
SITER = EPT_S // (2 * SCHUNK)
NPT = N_NODES // NS
FH = 32


@functools.lru_cache(maxsize=1)
def _mesh():
    return plsc.VectorSubcoreMesh(core_axis_name="c", subcore_axis_name="s",
                                  num_cores=NC, num_subcores=NS)


def _gather_body(h_hbm, s2_hbm, e2_hbm, hs_hbm, he_hbm, idx_v, rows_v, sem):
    c = lax.axis_index("c")
    s = lax.axis_index("s")
    wid = s * NC + c
    for idx_hbm, out_hbm in ((s2_hbm, hs_hbm), (e2_hbm, he_hbm)):
        def chunk(i, carry, idx_hbm=idx_hbm, out_hbm=out_hbm):
            rbase = wid * RPW_G + i * GROWS
            ebase = wid * EPW_G + i * GCHUNK
            pltpu.sync_copy(idx_hbm.at[pl.ds(rbase, GROWS)], idx_v)
            cps = [
                pltpu.async_copy(h_hbm.at[idx_v.at[j]],
                                 rows_v.at[pl.ds(j * IROW, IROW)], sem)
                for j in range(GROWS)
            ]
            for cp in cps:
                cp.wait()
            pltpu.sync_copy(rows_v, out_hbm.at[pl.ds(ebase, GCHUNK)])
            return carry
        lax.fori_loop(0, EPW_G // GCHUNK, chunk, 0)


def _sc_gather(h, s2, e2):
    return pl.kernel(
        _gather_body,
        out_type=[jax.ShapeDtypeStruct((N_EDGES, HPAD), jnp.float32),
                  jax.ShapeDtypeStruct((N_EDGES, HPAD), jnp.float32)],
        mesh=_mesh(),
        scratch_types=[
            pltpu.VMEM((GROWS, IROW), jnp.int32),
            pltpu.VMEM((GCHUNK, HPAD), jnp.float32),
            pltpu.SemaphoreType.DMA,
        ],
    )(h, s2, e2)


def _scatter_body(e_hbm, s2_hbm, e2_hbm, z_hbm, agg_e_hbm, agg_s_hbm,
                  idx0, idx1, val0, val1, acc, sl0, sl1, ss0, ss1):
    c = lax.axis_index("c")
    s = lax.axis_index("s")
    rb = s * NPT
    slots = ((idx0, val0, sl0, ss0), (idx1, val1, sl1, ss1))

    def do_half(fh):
        for idx_hbm, agg_hbm in ((e2_hbm, agg_e_hbm), (s2_hbm, agg_s_hbm)):
            pltpu.sync_copy(z_hbm.at[pl.ds(rb, NPT)], acc.at[pl.ds(rb, NPT)])
            plsc.subcore_barrier()

            def step(g, carry, idx_hbm=idx_hbm):
                for b, (idx_v, val_v, sl, ss) in enumerate(slots):
                    i = 2 * g + b
                    rbase = s * RPT_S + i * SROWS
                    ebase = s * EPT_S + i * SCHUNK

                    @pl.when(g > 0)
                    def _(idx_v=idx_v, val_v=val_v, ss=ss):
                        for j in range(SROWS):
                            pltpu.make_async_copy(
                                val_v.at[pl.ds(j * IROW, IROW)],
                                acc.at[idx_v.at[j]], ss).wait()

                    pltpu.async_copy(idx_hbm.at[pl.ds(rbase, SROWS)],
                                     idx_v, sl)
                    pltpu.async_copy(
                        e_hbm.at[pl.ds(ebase, SCHUNK), pl.ds(fh, FH)],
                        val_v, sl)
                for b, (idx_v, val_v, sl, ss) in enumerate(slots):
                    pltpu.make_async_copy(
                        idx_hbm.at[pl.ds(s * RPT_S, SROWS)], idx_v, sl).wait()
                    pltpu.make_async_copy(
                        e_hbm.at[pl.ds(s * EPT_S, SCHUNK), pl.ds(fh, FH)],
                        val_v, sl).wait()
                    for j in range(SROWS):
                        pltpu.async_copy(val_v.at[pl.ds(j * IROW, IROW)],
                                         acc.at[idx_v.at[j]], ss, add=True)
                return carry
            lax.fori_loop(0, SITER, step, 0)
            for b, (idx_v, val_v, sl, ss) in enumerate(slots):
                for j in range(SROWS):
                    pltpu.make_async_copy(
                        val_v.at[pl.ds(j * IROW, IROW)],
                        acc.at[idx_v.at[j]], ss).wait()
            plsc.subcore_barrier()
            pltpu.sync_copy(acc.at[pl.ds(rb, NPT)],
                            agg_hbm.at[pl.ds(rb, NPT), pl.ds(fh, FH)])
            plsc.subcore_barrier()

    for cc in range(NC):
        @pl.when(c == cc)
        def _(cc=cc):
            do_half(cc * FH)


def _sc_segment_sums(e, s2, e2, z32):
    return pl.kernel(
        _scatter_body,
        out_type=[jax.ShapeDtypeStruct((N_NODES, HIDDEN), jnp.float32),
                  jax.ShapeDtypeStruct((N_NODES, HIDDEN), jnp.float32)],
        mesh=_mesh(),
        scratch_types=[
            pltpu.VMEM((SROWS, IROW), jnp.int32),
            pltpu.VMEM((SROWS, IROW), jnp.int32),
            pltpu.VMEM((SCHUNK, FH), jnp.float32),
            pltpu.VMEM((SCHUNK, FH), jnp.float32),
            pltpu.VMEM_SHARED((N_NODES, FH), jnp.float32),
            pltpu.SemaphoreType.DMA,
            pltpu.SemaphoreType.DMA,
            pltpu.SemaphoreType.DMA,
            pltpu.SemaphoreType.DMA,
        ],
        compiler_params=pltpu.CompilerParams(use_tc_tiling_on_sc=False),
    )(e, s2, e2, z32)


def _tc_mlp(inputs, layers, use_dims, acts, block_rows, n_rows,
            out_pad=None):
    flat = []
    for lp in layers:
        flat.append(lp[0])
        flat.append(lp[1].reshape(1, -1))
        if len(lp) == 4:
            flat.append(lp[2].reshape(1, -1))
            flat.append(lp[3].reshape(1, -1))
    out_dim = layers[-1][0].shape[1]
    phys_dims = [a.shape[1] for a in inputs]
    n_in = len(inputs)
    out_phys = out_pad if out_pad is not None else out_dim

    def body(*refs):
        irefs = refs[:n_in]
        wrefs = refs[n_in:-1]
        oref = refs[-1]
        wi = 0
        xcur = None
        for li, lp in enumerate(layers):
            w = wrefs[wi][...]
            b = wrefs[wi + 1][...]
            wi += 2
            if li == 0:
                off = 0
                z = None
                for k, ir in enumerate(irefs):
                    xk = ir[...][:, :use_dims[k]]
                    t = jnp.dot(xk, w[off:off + use_dims[k], :],
                                preferred_element_type=jnp.float32)
                    z = t if z is None else z + t
                    off += use_dims[k]
                z = z + b
            else:
                z = jnp.dot(xcur, w, preferred_element_type=jnp.float32) + b
            if len(lp) == 4:
                g = wrefs[wi][...]
                bt = wrefs[wi + 1][...]
                wi += 2
                mu = jnp.mean(z, axis=-1, keepdims=True)
                var = jnp.mean((z - mu) ** 2, axis=-1, keepdims=True)
                z = (z - mu) * lax.rsqrt(var + 1e-5) * g + bt
            if acts[li] == 'relu':
                z = jnp.maximum(z, 0.0)
            elif acts[li] == 'tanh':
                z = jnp.tanh(z)
            xcur = z
        if out_phys > out_dim:
            pad = jnp.zeros((xcur.shape[0], out_phys - out_dim), jnp.float32)
            xcur = jnp.concatenate([xcur, pad], axis=-1)
        oref[...] = xcur

    grid = (n_rows // block_rows,)
    in_specs = (
        [pl.BlockSpec((block_rows, d), lambda i: (i, 0)) for d in phys_dims]
        + [pl.BlockSpec(w.shape, lambda i: (0,) * w.ndim) for w in flat]
    )
    return pl.pallas_call(
        body,
        grid=grid,
        in_specs=in_specs,
        out_specs=pl.BlockSpec((block_rows, out_phys), lambda i: (i, 0)),
        out_shape=jax.ShapeDtypeStruct((n_rows, out_phys), jnp.float32),
        compiler_params=pltpu.CompilerParams(
            dimension_semantics=("arbitrary",)),
    )(*inputs, *flat)


B_NODE = 2000
B_EDGE = 8000


def kernel(x, edge_index, node_enc, edge_enc, node_net, edge_net, edge_clf):
    start = edge_index[0]
    end = edge_index[1]
    s2 = start.reshape(NROWS, IROW)
    e2 = end.reshape(NROWS, IROW)
    z32 = jnp.zeros((N_NODES, FH), jnp.float32)

    h = _tc_mlp([x], node_enc, [3], ['relu', 'tanh'], B_NODE, N_NODES,
                out_pad=HPAD)
    hs, he = _sc_gather(h, s2, e2)
    e = _tc_mlp([hs, he], edge_enc, [HIDDEN, HIDDEN], ['relu', 'tanh'],
                B_EDGE, N_EDGES)
    for _ in range(3):
        agg_e, agg_s = _sc_segment_sums(e, s2, e2, z32)
        h = _tc_mlp([h, agg_e, agg_s], node_net, [HIDDEN] * 3,
                    ['relu', 'tanh'], B_NODE, N_NODES, out_pad=HPAD)
        hs, he = _sc_gather(h, s2, e2)
        e = _tc_mlp([hs, he, e], edge_net, [HIDDEN] * 3, ['relu', 'tanh'],
                    B_EDGE, N_EDGES)
    out = _tc_mlp([hs, he, e], edge_clf, [HIDDEN] * 3,
                  ['relu', 'relu', None], B_EDGE, N_EDGES)
    return jnp.squeeze(out, axis=-1)
